# baseline (speedup 1.0000x reference)
import functools

import jax
import jax.numpy as jnp
from jax import lax
from jax.experimental import pallas as pl
from jax.experimental.pallas import tpu as pltpu

S = 512
D = 128
_TPAD = 1024


def _tc_body(table_ref, out_ref):
    i = pl.program_id(0)
    out_ref[0] = table_ref[pl.ds(S - 1 - i, S), :]


def kernel(rel_pos_embedding, shifted_positions):
    del shifted_positions
    table = jnp.pad(rel_pos_embedding, ((0, _TPAD - (2 * S - 1)), (0, 0)))
    out = pl.pallas_call(
        _tc_body,
        grid=(S,),
        in_specs=[pl.BlockSpec((_TPAD, D), lambda i: (0, 0))],
        out_specs=pl.BlockSpec((1, S, D), lambda i: (i, 0, 0)),
        out_shape=jax.ShapeDtypeStruct((S, S, D), jnp.float32),
    )(table)
    return out


# speedup vs baseline: 3.7777x; 3.7777x over previous
import functools

import jax
import jax.numpy as jnp
from jax import lax
from jax.experimental import pallas as pl
from jax.experimental.pallas import tpu as pltpu

S = 512
D = 128
_TPAD = 1024


_RB = 8


def _tc_body(table_ref, out_ref):
    i = pl.program_id(0)
    for r in range(_RB):
        out_ref[r] = table_ref[pl.ds(S - 1 - (i * _RB + r), S), :]


def kernel(rel_pos_embedding, shifted_positions):
    del shifted_positions
    table = jnp.pad(rel_pos_embedding, ((0, _TPAD - (2 * S - 1)), (0, 0)))
    out = pl.pallas_call(
        _tc_body,
        grid=(S // _RB,),
        in_specs=[pl.BlockSpec((_TPAD, D), lambda i: (0, 0))],
        out_specs=pl.BlockSpec((_RB, S, D), lambda i: (i, 0, 0)),
        out_shape=jax.ShapeDtypeStruct((S, S, D), jnp.float32),
    )(table)
    return out


# speedup vs baseline: 4.3663x; 1.1558x over previous
import functools

import jax
import jax.numpy as jnp
from jax import lax
from jax.experimental import pallas as pl
from jax.experimental.pallas import tpu as pltpu

S = 512
D = 128
_TPAD = 1024


_RB = 16


def _tc_body(table_ref, out_ref):
    i = pl.program_id(0)
    for r in range(_RB):
        out_ref[r] = table_ref[pl.ds(S - 1 - (i * _RB + r), S), :]


def kernel(rel_pos_embedding, shifted_positions):
    del shifted_positions
    table = jnp.pad(rel_pos_embedding, ((0, _TPAD - (2 * S - 1)), (0, 0)))
    out = pl.pallas_call(
        _tc_body,
        grid=(S // _RB,),
        in_specs=[pl.BlockSpec((_TPAD, D), lambda i: (0, 0))],
        out_specs=pl.BlockSpec((_RB, S, D), lambda i: (i, 0, 0)),
        out_shape=jax.ShapeDtypeStruct((S, S, D), jnp.float32),
    )(table)
    return out
